# Initial kernel scaffold; baseline (speedup 1.0000x reference)
import jax
import jax.numpy as jnp
from jax.experimental import pallas as pl


def kernel(item_tag1, item_tag2, item_tag3, user_tag0, emb_single, emb_pair_a, emb_pair_b, W1, b1, g1, beta1, a1, W2, b2, g2, beta2, a2, W3, b3):
    raise NotImplementedError("write your pallas kernel here")


# speedup vs baseline: 1.1572x; 1.1572x over previous
import functools

import jax
import jax.numpy as jnp
from jax import lax
from jax.experimental import pallas as pl
from jax.experimental.pallas import tpu as pltpu
from jax.experimental.pallas import tpu_sc as plsc

_B = 16384
_V = 100000
_D = 16
_PAIRS = ((0, 1), (0, 2), (0, 3), (1, 2), (1, 3), (2, 3))

_NC = 2
_NS = 16
_NW = _NC * _NS
_BPW = _B // _NW

_EPS = 1e-3


def _sc_gather_call(idx, emb_single, emb_a_flat, emb_b_flat):
    mesh = plsc.VectorSubcoreMesh(core_axis_name="c", subcore_axis_name="s")

    @functools.partial(
        pl.kernel,
        out_type=jax.ShapeDtypeStruct((16, _B, _D), jnp.float32),
        mesh=mesh,
        compiler_params=pltpu.CompilerParams(use_tc_tiling_on_sc=False),
        scratch_types=[
            pltpu.VMEM((16, _BPW), jnp.int32),
            pltpu.VMEM((2, _BPW, _D), jnp.float32),
            pltpu.SemaphoreType.DMA,
            pltpu.SemaphoreType.DMA,
        ],
    )
    def sc_kernel(idx_hbm, single_hbm, a_hbm, b_hbm, out_hbm, idx_v, buf_v, gsem, wsem):
        wid = lax.axis_index("s") * _NC + lax.axis_index("c")
        base = wid * _BPW
        pltpu.sync_copy(idx_hbm.at[wid], idx_v)

        tables = [single_hbm] * 4 + [a_hbm] * 6 + [b_hbm] * 6

        copy0 = pltpu.async_copy(
            tables[0].at[idx_v.at[0]], buf_v.at[0], gsem
        )
        prev = copy0
        for f in range(16):
            nxt = None
            if f + 1 < 16:
                nxt = pltpu.async_copy(
                    tables[f + 1].at[idx_v.at[f + 1]], buf_v.at[(f + 1) % 2], gsem
                )
            prev.wait()
            wcopy = pltpu.async_copy(
                buf_v.at[f % 2], out_hbm.at[f, pl.ds(base, _BPW)], wsem
            )
            wcopy.wait()
            prev = nxt

    return sc_kernel(idx, emb_single, emb_a_flat, emb_b_flat)


def _mlp_body(feat_ref, w1_ref, b1_ref, g1_ref, be1_ref, a1_ref,
              w2_ref, b2_ref, g2_ref, be2_ref, a2_ref,
              w3_ref, b3_ref, out_ref):
    blk = feat_ref.shape[1]
    acc = jnp.zeros((blk, 40), jnp.float32)
    for f in range(4):
        acc += jnp.dot(feat_ref[f], w1_ref[f],
                       preferred_element_type=jnp.float32,
                       precision=lax.Precision.HIGHEST)
    for p in range(6):
        prod = feat_ref[4 + p] * feat_ref[10 + p]
        acc += jnp.dot(prod, w1_ref[4 + p],
                       preferred_element_type=jnp.float32,
                       precision=lax.Precision.HIGHEST)
    h = acc + b1_ref[...]
    mu = jnp.mean(h, axis=-1, keepdims=True)
    var = jnp.mean(jnp.square(h - mu), axis=-1, keepdims=True)
    h = (h - mu) * lax.rsqrt(var + _EPS) * g1_ref[...] + be1_ref[...]
    h = jnp.where(h >= 0, h, a1_ref[...] * h)

    h = jnp.dot(h, w2_ref[...],
                preferred_element_type=jnp.float32,
                precision=lax.Precision.HIGHEST) + b2_ref[...]
    mu = jnp.mean(h, axis=-1, keepdims=True)
    var = jnp.mean(jnp.square(h - mu), axis=-1, keepdims=True)
    h = (h - mu) * lax.rsqrt(var + _EPS) * g2_ref[...] + be2_ref[...]
    h = jnp.where(h >= 0, h, a2_ref[...] * h)

    logit = jnp.sum(h * w3_ref[...], axis=-1, keepdims=True) + b3_ref[...]
    out_ref[...] = jax.nn.sigmoid(logit)


def _tc_mlp_call(feats, w1r, b1, g1, be1, a1, w2, b2, g2, be2, a2, w3t, b3):
    blk = 1024
    grid = _B // blk
    full = lambda shape: pl.BlockSpec(shape, lambda i: (0,) * len(shape))
    return pl.pallas_call(
        _mlp_body,
        grid=(grid,),
        in_specs=[
            pl.BlockSpec((16, blk, _D), lambda i: (0, i, 0)),
            full((10, _D, 40)),
            full((1, 40)), full((1, 40)), full((1, 40)), full((1, 40)),
            full((40, 20)),
            full((1, 20)), full((1, 20)), full((1, 20)), full((1, 20)),
            full((1, 20)), full((1, 1)),
        ],
        out_specs=pl.BlockSpec((blk, 1), lambda i: (i, 0)),
        out_shape=jax.ShapeDtypeStruct((_B, 1), jnp.float32),
    )(feats, w1r, b1, g1, be1, a1, w2, b2, g2, be2, a2, w3t, b3)


def kernel(item_tag1, item_tag2, item_tag3, user_tag0,
           emb_single, emb_pair_a, emb_pair_b,
           W1, b1, g1, beta1, a1,
           W2, b2, g2, beta2, a2,
           W3, b3):
    cols = [item_tag1[:, 0], item_tag2[:, 0], item_tag3[:, 0], user_tag0[:, 0]]
    rows = list(cols)
    for p, (i, j) in enumerate(_PAIRS):
        rows.append(cols[i] + p * _V)
    for p, (i, j) in enumerate(_PAIRS):
        rows.append(cols[j] + p * _V)
    idx = jnp.stack(rows)
    idx = idx.reshape(16, _NW, _BPW).transpose(1, 0, 2)

    feats = _sc_gather_call(
        idx,
        emb_single,
        emb_pair_a.reshape(6 * _V, _D),
        emb_pair_b.reshape(6 * _V, _D),
    )

    return _tc_mlp_call(
        feats,
        W1.reshape(10, _D, 40),
        b1.reshape(1, 40), g1.reshape(1, 40),
        beta1.reshape(1, 40), a1.reshape(1, 40),
        W2,
        b2.reshape(1, 20), g2.reshape(1, 20),
        beta2.reshape(1, 20), a2.reshape(1, 20),
        W3.reshape(1, 20), b3.reshape(1, 1),
    )


# speedup vs baseline: 1.1583x; 1.0009x over previous
import functools

import jax
import jax.numpy as jnp
from jax import lax
from jax.experimental import pallas as pl
from jax.experimental.pallas import tpu as pltpu
from jax.experimental.pallas import tpu_sc as plsc

_B = 16384
_V = 100000
_D = 16
_PAIRS = ((0, 1), (0, 2), (0, 3), (1, 2), (1, 3), (2, 3))

_NC = 2
_NS = 16
_NW = _NC * _NS
_BPW = _B // _NW

_EPS = 1e-3


def _sc_gather_call(t1, t2, t3, t0, emb_single, emb_a, emb_b):
    mesh = plsc.VectorSubcoreMesh(core_axis_name="c", subcore_axis_name="s")

    @functools.partial(
        pl.kernel,
        out_type=jax.ShapeDtypeStruct((16, _B, _D), jnp.float32),
        mesh=mesh,
        compiler_params=pltpu.CompilerParams(use_tc_tiling_on_sc=False),
        scratch_types=[
            pltpu.VMEM((4, _BPW), jnp.int32),
            pltpu.VMEM((2, _BPW, _D), jnp.float32),
            pltpu.SemaphoreType.DMA,
            pltpu.SemaphoreType.DMA,
        ],
    )
    def sc_kernel(t1_hbm, t2_hbm, t3_hbm, t0_hbm, single_hbm, a_hbm, b_hbm,
                  out_hbm, idx_v, buf_v, gsem, wsem):
        wid = lax.axis_index("s") * _NC + lax.axis_index("c")
        base = wid * _BPW
        for t, tag in enumerate((t1_hbm, t2_hbm, t3_hbm, t0_hbm)):
            pltpu.sync_copy(tag.at[pl.ds(base, _BPW)], idx_v.at[t])

        srcs = [(single_hbm, t) for t in range(4)]
        for p, (i, j) in enumerate(_PAIRS):
            srcs.append((a_hbm.at[p], i))
        for p, (i, j) in enumerate(_PAIRS):
            srcs.append((b_hbm.at[p], j))

        tbl0, row0 = srcs[0]
        prev = pltpu.async_copy(tbl0.at[idx_v.at[row0]], buf_v.at[0], gsem)
        for f in range(16):
            nxt = None
            if f + 1 < 16:
                tbl, row = srcs[f + 1]
                nxt = pltpu.async_copy(
                    tbl.at[idx_v.at[row]], buf_v.at[(f + 1) % 2], gsem
                )
            prev.wait()
            wcopy = pltpu.async_copy(
                buf_v.at[f % 2], out_hbm.at[f, pl.ds(base, _BPW)], wsem
            )
            wcopy.wait()
            prev = nxt

    return sc_kernel(t1, t2, t3, t0, emb_single, emb_a, emb_b)


def _mlp_body(feat_ref, w1_ref, b1_ref, g1_ref, be1_ref, a1_ref,
              w2_ref, b2_ref, g2_ref, be2_ref, a2_ref,
              w3_ref, b3_ref, out_ref):
    blk = feat_ref.shape[1]
    acc = jnp.zeros((blk, 40), jnp.float32)
    for f in range(4):
        acc += jnp.dot(feat_ref[f], w1_ref[f],
                       preferred_element_type=jnp.float32,
                       precision=lax.Precision.HIGHEST)
    for p in range(6):
        prod = feat_ref[4 + p] * feat_ref[10 + p]
        acc += jnp.dot(prod, w1_ref[4 + p],
                       preferred_element_type=jnp.float32,
                       precision=lax.Precision.HIGHEST)
    h = acc + b1_ref[...]
    mu = jnp.mean(h, axis=-1, keepdims=True)
    var = jnp.mean(jnp.square(h - mu), axis=-1, keepdims=True)
    h = (h - mu) * lax.rsqrt(var + _EPS) * g1_ref[...] + be1_ref[...]
    h = jnp.where(h >= 0, h, a1_ref[...] * h)

    h = jnp.dot(h, w2_ref[...],
                preferred_element_type=jnp.float32,
                precision=lax.Precision.HIGHEST) + b2_ref[...]
    mu = jnp.mean(h, axis=-1, keepdims=True)
    var = jnp.mean(jnp.square(h - mu), axis=-1, keepdims=True)
    h = (h - mu) * lax.rsqrt(var + _EPS) * g2_ref[...] + be2_ref[...]
    h = jnp.where(h >= 0, h, a2_ref[...] * h)

    logit = jnp.sum(h * w3_ref[...], axis=-1, keepdims=True) + b3_ref[...]
    out_ref[...] = jax.nn.sigmoid(logit)


def _tc_mlp_call(feats, w1r, b1, g1, be1, a1, w2, b2, g2, be2, a2, w3t, b3):
    blk = 1024
    grid = _B // blk
    full = lambda shape: pl.BlockSpec(shape, lambda i: (0,) * len(shape))
    return pl.pallas_call(
        _mlp_body,
        grid=(grid,),
        in_specs=[
            pl.BlockSpec((16, blk, _D), lambda i: (0, i, 0)),
            full((10, _D, 40)),
            full((1, 40)), full((1, 40)), full((1, 40)), full((1, 40)),
            full((40, 20)),
            full((1, 20)), full((1, 20)), full((1, 20)), full((1, 20)),
            full((1, 20)), full((1, 1)),
        ],
        out_specs=pl.BlockSpec((blk, 1), lambda i: (i, 0)),
        out_shape=jax.ShapeDtypeStruct((_B, 1), jnp.float32),
    )(feats, w1r, b1, g1, be1, a1, w2, b2, g2, be2, a2, w3t, b3)


def kernel(item_tag1, item_tag2, item_tag3, user_tag0,
           emb_single, emb_pair_a, emb_pair_b,
           W1, b1, g1, beta1, a1,
           W2, b2, g2, beta2, a2,
           W3, b3):
    feats = _sc_gather_call(
        item_tag1.reshape(_B), item_tag2.reshape(_B),
        item_tag3.reshape(_B), user_tag0.reshape(_B),
        emb_single, emb_pair_a, emb_pair_b,
    )

    return _tc_mlp_call(
        feats,
        W1.reshape(10, _D, 40),
        b1.reshape(1, 40), g1.reshape(1, 40),
        beta1.reshape(1, 40), a1.reshape(1, 40),
        W2,
        b2.reshape(1, 20), g2.reshape(1, 20),
        beta2.reshape(1, 20), a2.reshape(1, 20),
        W3.reshape(1, 20), b3.reshape(1, 1),
    )


# speedup vs baseline: 1.3738x; 1.1860x over previous
import functools

import jax
import jax.numpy as jnp
from jax import lax
from jax.experimental import pallas as pl
from jax.experimental.pallas import tpu as pltpu
from jax.experimental.pallas import tpu_sc as plsc

_B = 16384
_V = 100000
_D = 16
_PAIRS = ((0, 1), (0, 2), (0, 3), (1, 2), (1, 3), (2, 3))

_NC = 2
_NS = 16
_NW = _NC * _NS
_BPW = _B // _NW

_EPS = 1e-3


def _sc_gather_call(t1, t2, t3, t0, emb_single, emb_a, emb_b):
    mesh = plsc.VectorSubcoreMesh(core_axis_name="c", subcore_axis_name="s")

    @functools.partial(
        pl.kernel,
        out_type=jax.ShapeDtypeStruct((16, _B, _D), jnp.float32),
        mesh=mesh,
        compiler_params=pltpu.CompilerParams(use_tc_tiling_on_sc=False),
        scratch_types=[
            pltpu.VMEM((4, _BPW), jnp.int32),
            pltpu.VMEM((2, _BPW, _D), jnp.float32),
            pltpu.SemaphoreType.DMA,
            pltpu.SemaphoreType.DMA,
        ],
    )
    def sc_kernel(t1_hbm, t2_hbm, t3_hbm, t0_hbm, single_hbm, a_hbm, b_hbm,
                  out_hbm, idx_v, buf_v, gsem, wsem):
        wid = lax.axis_index("s") * _NC + lax.axis_index("c")
        base = wid * _BPW
        for t, tag in enumerate((t1_hbm, t2_hbm, t3_hbm, t0_hbm)):
            pltpu.sync_copy(tag.at[pl.ds(base, _BPW)], idx_v.at[t])

        srcs = [(single_hbm, t) for t in range(4)]
        for p, (i, j) in enumerate(_PAIRS):
            srcs.append((a_hbm.at[p], i))
        for p, (i, j) in enumerate(_PAIRS):
            srcs.append((b_hbm.at[p], j))

        tbl0, row0 = srcs[0]
        prev = pltpu.async_copy(tbl0.at[idx_v.at[row0]], buf_v.at[0], gsem)
        for f in range(16):
            nxt = None
            if f + 1 < 16:
                tbl, row = srcs[f + 1]
                nxt = pltpu.async_copy(
                    tbl.at[idx_v.at[row]], buf_v.at[(f + 1) % 2], gsem
                )
            prev.wait()
            wcopy = pltpu.async_copy(
                buf_v.at[f % 2], out_hbm.at[f, pl.ds(base, _BPW)], wsem
            )
            wcopy.wait()
            prev = nxt

    return sc_kernel(t1, t2, t3, t0, emb_single, emb_a, emb_b)


def _dot(a, b):
    return jnp.dot(a, b, preferred_element_type=jnp.float32,
                   precision=lax.Precision.HIGHEST)


def _mlp_body(feat_ref, w1_ref, m1_ref, b1_ref, g1_ref, be1_ref, a1_ref,
              w2_ref, m2_ref, b2_ref, g2_ref, be2_ref, a2_ref,
              w3_ref, b3_ref, out_ref):
    blk8 = feat_ref.shape[1]
    acc = jnp.zeros((blk8, 320), jnp.float32)
    for f in range(4):
        acc += _dot(feat_ref[f], w1_ref[f])
    for p in range(6):
        acc += _dot(feat_ref[4 + p] * feat_ref[10 + p], w1_ref[4 + p])
    h = acc + b1_ref[...]
    mu = _dot(h, m1_ref[...])
    d = h - mu
    var = _dot(d * d, m1_ref[...])
    h = d * lax.rsqrt(var + _EPS) * g1_ref[...] + be1_ref[...]
    h = jnp.where(h >= 0, h, a1_ref[...] * h)

    h = _dot(h, w2_ref[...]) + b2_ref[...]
    mu = _dot(h, m2_ref[...])
    d = h - mu
    var = _dot(d * d, m2_ref[...])
    h = d * lax.rsqrt(var + _EPS) * g2_ref[...] + be2_ref[...]
    h = jnp.where(h >= 0, h, a2_ref[...] * h)

    logit = _dot(h, w3_ref[...]) + b3_ref[...]
    out_ref[...] = jax.nn.sigmoid(logit)


def _tc_mlp_call(feats8, w1big, m1, b1, g1, be1, a1,
                 w2big, m2, b2, g2, be2, a2, w3big, b3):
    blk8 = 256
    grid = (_B // 8) // blk8
    full = lambda shape: pl.BlockSpec(shape, lambda i: (0,) * len(shape))
    return pl.pallas_call(
        _mlp_body,
        grid=(grid,),
        in_specs=[
            pl.BlockSpec((16, blk8, 128), lambda i: (0, i, 0)),
            full((10, 128, 320)),
            full((320, 320)),
            full((1, 320)), full((1, 320)), full((1, 320)), full((1, 320)),
            full((320, 160)),
            full((160, 160)),
            full((1, 160)), full((1, 160)), full((1, 160)), full((1, 160)),
            full((160, 8)), full((1, 1)),
        ],
        out_specs=pl.BlockSpec((blk8, 8), lambda i: (i, 0)),
        out_shape=jax.ShapeDtypeStruct((_B // 8, 8), jnp.float32),
    )(feats8, w1big, m1, b1, g1, be1, a1, w2big, m2, b2, g2, be2, a2,
      w3big, b3)


def kernel(item_tag1, item_tag2, item_tag3, user_tag0,
           emb_single, emb_pair_a, emb_pair_b,
           W1, b1, g1, beta1, a1,
           W2, b2, g2, beta2, a2,
           W3, b3):
    feats = _sc_gather_call(
        item_tag1.reshape(_B), item_tag2.reshape(_B),
        item_tag3.reshape(_B), user_tag0.reshape(_B),
        emb_single, emb_pair_a, emb_pair_b,
    )
    feats8 = feats.reshape(16, _B // 8, 128)

    eye8 = jnp.eye(8, dtype=jnp.float32)
    kron = lambda w: jnp.einsum("ab,do->adbo", eye8, w).reshape(
        8 * w.shape[0], 8 * w.shape[1])
    w1big = jnp.stack([kron(W1.reshape(10, _D, 40)[f]) for f in range(10)])
    m1 = kron(jnp.full((40, 40), 1.0 / 40, jnp.float32))
    m2 = kron(jnp.full((20, 20), 1.0 / 20, jnp.float32))
    w2big = kron(W2)
    w3big = kron(W3)
    tile8 = lambda v: jnp.tile(v, 8).reshape(1, 8 * v.shape[0])

    out8 = _tc_mlp_call(
        feats8,
        w1big, m1,
        tile8(b1), tile8(g1), tile8(beta1), tile8(a1),
        w2big, m2,
        tile8(b2), tile8(g2), tile8(beta2), tile8(a2),
        w3big, b3.reshape(1, 1),
    )
    return out8.reshape(_B, 1)
